# Initial kernel scaffold; baseline (speedup 1.0000x reference)
import jax
import jax.numpy as jnp
from jax.experimental import pallas as pl


def kernel(x, edge_index_1, edge_index_2, index_1, index_2, Wi1, bi1, Wi2, bi2, Wc11, bc11, Wc12, bc12, Wm1a, bm1a, Wm1b, bm1b, Wc21, bc21, Wc22, bc22, Wm2a, bm2a, Wm2b, bm2b, Wfa, bfa, Wfb, bfb):
    raise NotImplementedError("write your pallas kernel here")


# speedup vs baseline: 4.6597x; 4.6597x over previous
import functools

import jax
import jax.numpy as jnp
from jax import lax
from jax.experimental import pallas as pl
from jax.experimental.pallas import tpu as pltpu
from jax.experimental.pallas import tpu_sc as plsc

N_T = 10000
N_O = 1000
E = 160000
DIM = 256
FIN = 4652
HALF = 128

NC = 2
NS = 16
LANES = 16

CHUNK = 128
NSUB = 8
E_PAD = NS * CHUNK * 80
N_PAD = NS * CHUNK * NSUB

ACC_NT = 10240
ACC_NO = 1024
ZROWS = ACC_NT // NS


def _sc_mesh():
    return plsc.VectorSubcoreMesh(
        core_axis_name="c", subcore_axis_name="s", num_cores=NC,
        num_subcores=NS)


def _hist_body(col1, col2, idx1, idx2, ones_hbm, z128_hbm,
               deg1_out, deg2_out, cnt1_out, cnt2_out,
               acc_deg, acc_cnt, ones_v, zb, ib):
    c = lax.axis_index("c")
    t = lax.axis_index("s")

    pltpu.sync_copy(ones_hbm, ones_v)
    pltpu.sync_copy(z128_hbm, zb)
    for m in range(ZROWS // CHUNK):
        pltpu.sync_copy(zb, acc_deg.at[pl.ds(t * ZROWS + m * CHUNK, CHUNK)])
    pltpu.sync_copy(zb.at[pl.ds(0, ACC_NO // NS)],
                    acc_cnt.at[pl.ds(t * (ACC_NO // NS), ACC_NO // NS)])
    plsc.subcore_barrier()

    def _accum(src2d, acc, nchunks):
        def body(j, _):
            pltpu.sync_copy(src2d.at[pl.ds(t * nchunks + j * NSUB, NSUB)],
                            ib)
            for sub in range(NSUB):
                pltpu.sync_copy(ones_v, acc.at[ib.at[sub]], add=True)
            return 0
        lax.fori_loop(0, nchunks // NSUB, body, 0)

    @pl.when(c == 0)
    def _():
        _accum(col1, acc_deg, E_PAD // (NS * CHUNK))
        _accum(idx1, acc_cnt, N_PAD // (NS * CHUNK))

    @pl.when(c == 1)
    def _():
        _accum(col2, acc_deg, E_PAD // (NS * CHUNK))
        _accum(idx2, acc_cnt, N_PAD // (NS * CHUNK))

    plsc.subcore_barrier()

    def _dump(acc, out, base_rows, nch, active):
        @pl.when(active)
        def _():
            for m in range(nch):
                sl = pl.ds(base_rows + m * CHUNK, CHUNK)
                pltpu.sync_copy(acc.at[sl], zb)
                pltpu.sync_copy(zb, out.at[sl])

    @pl.when(c == 0)
    def _():
        _dump(acc_deg, deg1_out, t * ZROWS, ZROWS // CHUNK, t >= 0)
        _dump(acc_cnt, cnt1_out, t * CHUNK, 1, t < 8)

    @pl.when(c == 1)
    def _():
        _dump(acc_deg, deg2_out, t * ZROWS, ZROWS // CHUNK, t >= 0)
        _dump(acc_cnt, cnt2_out, t * CHUNK, 1, t < 8)


def _sc_hist(col1, col2, idx1, idx2, ones_hbm, z128_hbm):
    f = pl.kernel(
        _hist_body,
        out_type=[
            jax.ShapeDtypeStruct((ACC_NT, HALF), jnp.float32),
            jax.ShapeDtypeStruct((ACC_NT, HALF), jnp.float32),
            jax.ShapeDtypeStruct((ACC_NO, HALF), jnp.float32),
            jax.ShapeDtypeStruct((ACC_NO, HALF), jnp.float32),
        ],
        mesh=_sc_mesh(),
        scratch_types=[
            pltpu.VMEM_SHARED((ACC_NT, HALF), jnp.float32),
            pltpu.VMEM_SHARED((ACC_NO, HALF), jnp.float32),
            pltpu.VMEM((CHUNK, HALF), jnp.float32),
            pltpu.VMEM((CHUNK, HALF), jnp.float32),
            pltpu.VMEM((NSUB, CHUNK), jnp.int32),
        ],
    )
    return f(col1, col2, idx1, idx2, ones_hbm, z128_hbm)


def _edge_body(row1, col1, row2, col2, a1v, a2v, z128_hbm,
               scat1_out, scat2_out,
               acc, rows, ibr, ibc, gb, sem):
    c = lax.axis_index("c")
    t = lax.axis_index("s")
    nchunks = E_PAD // (NS * CHUNK)

    def run_list(row2d, col2d, a_hbm, out_hbm):
        pltpu.sync_copy(z128_hbm, rows)
        for m in range(ZROWS // CHUNK):
            pltpu.sync_copy(rows, acc.at[pl.ds(t * ZROWS + m * CHUNK,
                                               CHUNK)])
        plsc.subcore_barrier()

        def body(j, _):
            sl = pl.ds(t * nchunks + j * NSUB, NSUB)
            pltpu.sync_copy(row2d.at[sl], ibr)
            pltpu.sync_copy(col2d.at[sl], ibc)
            for sub in range(NSUB):
                for v in range(CHUNK // LANES):
                    r = ibr[sub, pl.ds(v * LANES, LANES)]
                    gb[0, pl.ds(v * LANES, LANES)] = r * 2 + c
                pltpu.async_copy(a_hbm.at[gb.at[0]], rows, sem).wait()
                pltpu.sync_copy(rows, acc.at[ibc.at[sub]], add=True)
            return 0

        lax.fori_loop(0, nchunks // NSUB, body, 0)
        plsc.subcore_barrier()

        for m in range(ZROWS // CHUNK):
            sl = pl.ds(t * ZROWS + m * CHUNK, CHUNK)
            pltpu.sync_copy(acc.at[sl], rows)
            pltpu.sync_copy(rows, out_hbm.at[c, sl])
        plsc.subcore_barrier()

    run_list(row1, col1, a1v, scat1_out)
    run_list(row2, col2, a2v, scat2_out)


def _sc_edge_scatter(row1, col1, row2, col2, a1v, a2v, z128_hbm):
    f = pl.kernel(
        _edge_body,
        out_type=[
            jax.ShapeDtypeStruct((NC, ACC_NT, HALF), jnp.float32),
            jax.ShapeDtypeStruct((NC, ACC_NT, HALF), jnp.float32),
        ],
        mesh=_sc_mesh(),
        scratch_types=[
            pltpu.VMEM_SHARED((ACC_NT, HALF), jnp.float32),
            pltpu.VMEM((CHUNK, HALF), jnp.float32),
            pltpu.VMEM((NSUB, CHUNK), jnp.int32),
            pltpu.VMEM((NSUB, CHUNK), jnp.int32),
            pltpu.VMEM((1, CHUNK), jnp.int32),
            pltpu.SemaphoreType.DMA,
        ],
    )
    return f(row1, col1, row2, col2, a1v, a2v, z128_hbm)


def _mean_body(idx1, idx2, hv, z128_hbm,
               sum1_out, sum2_out,
               acc1, acc2, rows, zb, ib1, ib2, gb, sem):
    c = lax.axis_index("c")
    t = lax.axis_index("s")
    zrows = ACC_NO // NS

    pltpu.sync_copy(z128_hbm.at[pl.ds(0, zrows)], zb)
    pltpu.sync_copy(zb, acc1.at[pl.ds(t * zrows, zrows)])
    pltpu.sync_copy(zb, acc2.at[pl.ds(t * zrows, zrows)])
    plsc.subcore_barrier()

    pltpu.sync_copy(idx1.at[pl.ds(t * NSUB, NSUB)], ib1)
    pltpu.sync_copy(idx2.at[pl.ds(t * NSUB, NSUB)], ib2)
    for sub in range(NSUB):
        base = (t * NSUB + sub) * CHUNK
        for v in range(CHUNK // LANES):
            n = base + v * LANES + lax.iota(jnp.int32, LANES)
            n = jnp.minimum(n, N_T - 1)
            gb[0, pl.ds(v * LANES, LANES)] = n * 2 + c
        pltpu.async_copy(hv.at[gb.at[0]], rows, sem).wait()
        pltpu.sync_copy(rows, acc1.at[ib1.at[sub]], add=True)
        pltpu.sync_copy(rows, acc2.at[ib2.at[sub]], add=True)
    plsc.subcore_barrier()

    @pl.when(t < 8)
    def _():
        sl = pl.ds(t * (ACC_NO // 8), ACC_NO // 8)
        pltpu.sync_copy(acc1.at[sl], rows)
        pltpu.sync_copy(rows, sum1_out.at[c, sl])

    @pl.when(t >= 8)
    def _():
        sl = pl.ds((t - 8) * (ACC_NO // 8), ACC_NO // 8)
        pltpu.sync_copy(acc2.at[sl], rows)
        pltpu.sync_copy(rows, sum2_out.at[c, sl])


def _sc_mean(idx1, idx2, hv, z128_hbm):
    f = pl.kernel(
        _mean_body,
        out_type=[
            jax.ShapeDtypeStruct((NC, ACC_NO, HALF), jnp.float32),
            jax.ShapeDtypeStruct((NC, ACC_NO, HALF), jnp.float32),
        ],
        mesh=_sc_mesh(),
        scratch_types=[
            pltpu.VMEM_SHARED((ACC_NO, HALF), jnp.float32),
            pltpu.VMEM_SHARED((ACC_NO, HALF), jnp.float32),
            pltpu.VMEM((CHUNK, HALF), jnp.float32),
            pltpu.VMEM((ACC_NO // NS, HALF), jnp.float32),
            pltpu.VMEM((NSUB, CHUNK), jnp.int32),
            pltpu.VMEM((NSUB, CHUNK), jnp.int32),
            pltpu.VMEM((1, CHUNK), jnp.int32),
            pltpu.SemaphoreType.DMA,
        ],
    )
    return f(idx1, idx2, hv, z128_hbm)


_RB = 400


def _mlp_in_tc(x_ref, w1_ref, b1_ref, w2_ref, b2_ref, o_ref):
    h = jnp.dot(x_ref[...], w1_ref[...], preferred_element_type=jnp.float32)
    h = jnp.maximum(h + b1_ref[...], 0.0)
    o_ref[...] = (jnp.dot(h, w2_ref[...], preferred_element_type=jnp.float32)
                  + b2_ref[...])


def _tc_mlp_in(x, w1, b1, w2, b2, interpret=False):
    grid = (N_T // _RB,)
    return pl.pallas_call(
        _mlp_in_tc,
        grid=grid,
        in_specs=[
            pl.BlockSpec((_RB, FIN), lambda i: (i, 0)),
            pl.BlockSpec((FIN, DIM), lambda i: (0, 0)),
            pl.BlockSpec((1, DIM), lambda i: (0, 0)),
            pl.BlockSpec((DIM, DIM), lambda i: (0, 0)),
            pl.BlockSpec((1, DIM), lambda i: (0, 0)),
        ],
        out_specs=pl.BlockSpec((_RB, DIM), lambda i: (i, 0)),
        out_shape=jax.ShapeDtypeStruct((N_T, DIM), jnp.float32),
        interpret=interpret,
    )(x, w1, b1, w2, b2)


def _branch_tc(h_ref, d1_ref, d2_ref, w1_ref, w2_ref, a1_ref, a2_ref):
    h = h_ref[...]
    dinv1 = lax.rsqrt(d1_ref[...][:, 0:1] + 1.0)
    dinv2 = lax.rsqrt(d2_ref[...][:, 0:1] + 1.0)
    a1_ref[...] = dinv1 * jnp.dot(h, w1_ref[...],
                                  preferred_element_type=jnp.float32)
    a2_ref[...] = dinv2 * jnp.dot(h, w2_ref[...],
                                  preferred_element_type=jnp.float32)


def _tc_branch(h, deg1, deg2, w1, w2, interpret=False):
    grid = (N_T // _RB,)
    return pl.pallas_call(
        _branch_tc,
        grid=grid,
        in_specs=[
            pl.BlockSpec((_RB, DIM), lambda i: (i, 0)),
            pl.BlockSpec((_RB, HALF), lambda i: (i, 0)),
            pl.BlockSpec((_RB, HALF), lambda i: (i, 0)),
            pl.BlockSpec((DIM, DIM), lambda i: (0, 0)),
            pl.BlockSpec((DIM, DIM), lambda i: (0, 0)),
        ],
        out_specs=[
            pl.BlockSpec((_RB, DIM), lambda i: (i, 0)),
            pl.BlockSpec((_RB, DIM), lambda i: (i, 0)),
        ],
        out_shape=[
            jax.ShapeDtypeStruct((N_T, DIM), jnp.float32),
            jax.ShapeDtypeStruct((N_T, DIM), jnp.float32),
        ],
        interpret=interpret,
    )(h, deg1, deg2, w1, w2)


def _combine_tc(s1l_ref, s1h_ref, s2l_ref, s2h_ref,
                a1l_ref, a1h_ref, a2l_ref, a2h_ref,
                d1_ref, d2_ref, bc1_ref, bc2_ref,
                wa_ref, ba_ref, wb_ref, bb_ref, o_ref):
    dinv1 = lax.rsqrt(d1_ref[...][:, 0:1] + 1.0)
    dinv2 = lax.rsqrt(d2_ref[...][:, 0:1] + 1.0)
    x1l = jnp.maximum(dinv1 * (s1l_ref[0] + a1l_ref[...])
                      + bc1_ref[...][:, :HALF], 0.0)
    x1h = jnp.maximum(dinv1 * (s1h_ref[0] + a1h_ref[...])
                      + bc1_ref[...][:, HALF:], 0.0)
    x2l = jnp.maximum(dinv2 * (s2l_ref[0] + a2l_ref[...])
                      + bc2_ref[...][:, :HALF], 0.0)
    x2h = jnp.maximum(dinv2 * (s2h_ref[0] + a2h_ref[...])
                      + bc2_ref[...][:, HALF:], 0.0)
    wa = wa_ref[...]
    pre = (jnp.dot(x1l, wa[0:HALF], preferred_element_type=jnp.float32)
           + jnp.dot(x1h, wa[HALF:DIM], preferred_element_type=jnp.float32)
           + jnp.dot(x2l, wa[DIM:DIM + HALF],
                     preferred_element_type=jnp.float32)
           + jnp.dot(x2h, wa[DIM + HALF:], preferred_element_type=jnp.float32)
           + ba_ref[...])
    pre = jnp.maximum(pre, 0.0)
    o_ref[...] = (jnp.dot(pre, wb_ref[...], preferred_element_type=jnp.float32)
                  + bb_ref[...])


def _tc_combine(s1, s2, a1, a2, deg1, deg2, bc1, bc2, wa, ba, wb, bb,
                interpret=False):
    grid = (N_T // _RB,)
    half_spec = [
        pl.BlockSpec((1, _RB, HALF), lambda i: (0, i, 0)),
        pl.BlockSpec((1, _RB, HALF), lambda i: (1, i, 0)),
    ]
    a_spec = [
        pl.BlockSpec((_RB, HALF), lambda i: (i, 0)),
        pl.BlockSpec((_RB, HALF), lambda i: (i, 1)),
    ]
    return pl.pallas_call(
        _combine_tc,
        grid=grid,
        in_specs=half_spec + half_spec + a_spec + a_spec + [
            pl.BlockSpec((_RB, HALF), lambda i: (i, 0)),
            pl.BlockSpec((_RB, HALF), lambda i: (i, 0)),
            pl.BlockSpec((1, DIM), lambda i: (0, 0)),
            pl.BlockSpec((1, DIM), lambda i: (0, 0)),
            pl.BlockSpec((2 * DIM, DIM), lambda i: (0, 0)),
            pl.BlockSpec((1, DIM), lambda i: (0, 0)),
            pl.BlockSpec((DIM, DIM), lambda i: (0, 0)),
            pl.BlockSpec((1, DIM), lambda i: (0, 0)),
        ],
        out_specs=pl.BlockSpec((_RB, DIM), lambda i: (i, 0)),
        out_shape=jax.ShapeDtypeStruct((N_T, DIM), jnp.float32),
        interpret=interpret,
    )(s1, s1, s2, s2, a1, a1, a2, a2, deg1, deg2, bc1, bc2, wa, ba, wb, bb)


_OUTW = 128


def _final_tc(m1l_ref, m1h_ref, m2l_ref, m2h_ref, c1_ref, c2_ref,
              wa_ref, ba_ref, wb_ref, bb_ref, o_ref):
    cnt1 = jnp.maximum(c1_ref[...][:, 0:1], 1.0)
    cnt2 = jnp.maximum(c2_ref[...][:, 0:1], 1.0)
    s1l = m1l_ref[0] / cnt1
    s1h = m1h_ref[0] / cnt1
    s2l = m2l_ref[0] / cnt2
    s2h = m2h_ref[0] / cnt2
    wa = wa_ref[...]
    pre = (jnp.dot(s1l, wa[0:HALF], preferred_element_type=jnp.float32)
           + jnp.dot(s1h, wa[HALF:DIM], preferred_element_type=jnp.float32)
           + jnp.dot(s2l, wa[DIM:DIM + HALF],
                     preferred_element_type=jnp.float32)
           + jnp.dot(s2h, wa[DIM + HALF:], preferred_element_type=jnp.float32)
           + ba_ref[...])
    pre = jnp.maximum(pre, 0.0)
    z = (jnp.dot(pre, wb_ref[...], preferred_element_type=jnp.float32)
         + bb_ref[...])
    lanes = lax.broadcasted_iota(jnp.int32, z.shape, 1)
    z = jnp.where(lanes < 5, z, -1e30)
    m = jnp.max(z, axis=1, keepdims=True)
    e = jnp.exp(z - m)
    s = jnp.sum(e, axis=1, keepdims=True)
    o_ref[...] = z - m - jnp.log(s)


def _tc_final(m1, m2, cnt1, cnt2, wa, ba, wb, bb, interpret=False):
    half_spec = [
        pl.BlockSpec((1, N_O, HALF), lambda i: (0, 0, 0)),
        pl.BlockSpec((1, N_O, HALF), lambda i: (1, 0, 0)),
    ]
    return pl.pallas_call(
        _final_tc,
        grid=(1,),
        in_specs=half_spec + half_spec + [
            pl.BlockSpec((N_O, HALF), lambda i: (0, 0)),
            pl.BlockSpec((N_O, HALF), lambda i: (0, 0)),
            pl.BlockSpec((2 * DIM, DIM), lambda i: (0, 0)),
            pl.BlockSpec((1, DIM), lambda i: (0, 0)),
            pl.BlockSpec((DIM, _OUTW), lambda i: (0, 0)),
            pl.BlockSpec((1, _OUTW), lambda i: (0, 0)),
        ],
        out_specs=pl.BlockSpec((N_O, _OUTW), lambda i: (0, 0)),
        out_shape=jax.ShapeDtypeStruct((N_O, _OUTW), jnp.float32),
        interpret=interpret,
    )(m1, m1, m2, m2, cnt1, cnt2, wa, ba, wb, bb)


def kernel(x, edge_index_1, edge_index_2, index_1, index_2, Wi1, bi1, Wi2,
           bi2, Wc11, bc11, Wc12, bc12, Wm1a, bm1a, Wm1b, bm1b, Wc21, bc21,
           Wc22, bc22, Wm2a, bm2a, Wm2b, bm2b, Wfa, bfa, Wfb, bfb):
    f32 = jnp.float32
    i32 = jnp.int32

    def pad_edges(ei):
        row = jnp.concatenate(
            [ei[0], jnp.zeros((E_PAD - E,), i32)]).reshape(E_PAD // CHUNK,
                                                           CHUNK)
        col = jnp.concatenate(
            [ei[1], jnp.full((E_PAD - E,), N_T, i32)]).reshape(
                E_PAD // CHUNK, CHUNK)
        return row, col

    row1, col1 = pad_edges(edge_index_1)
    row2, col2 = pad_edges(edge_index_2)
    idx1 = jnp.concatenate(
        [index_1, jnp.full((N_PAD - N_T,), N_O, i32)]).reshape(
            N_PAD // CHUNK, CHUNK)
    idx2 = jnp.concatenate(
        [index_2, jnp.full((N_PAD - N_T,), N_O, i32)]).reshape(
            N_PAD // CHUNK, CHUNK)

    ones_hbm = jnp.ones((CHUNK, HALF), f32)
    z128_hbm = jnp.zeros((CHUNK, HALF), f32)

    def r2(b):
        return b.reshape(1, DIM)

    deg1, deg2, cnt1, cnt2 = _sc_hist(col1, col2, idx1, idx2, ones_hbm,
                                      z128_hbm)

    h = _tc_mlp_in(x, Wi1, r2(bi1), Wi2, r2(bi2))

    a1, a2 = _tc_branch(h, deg1, deg2, Wc11, Wc12)
    s1, s2 = _sc_edge_scatter(row1, col1, row2, col2,
                              a1.reshape(2 * N_T, HALF),
                              a2.reshape(2 * N_T, HALF), z128_hbm)
    h = _tc_combine(s1, s2, a1, a2, deg1, deg2, r2(bc11), r2(bc12),
                    Wm1a, r2(bm1a), Wm1b, r2(bm1b))

    a1, a2 = _tc_branch(h, deg1, deg2, Wc21, Wc22)
    s1, s2 = _sc_edge_scatter(row1, col1, row2, col2,
                              a1.reshape(2 * N_T, HALF),
                              a2.reshape(2 * N_T, HALF), z128_hbm)
    h = _tc_combine(s1, s2, a1, a2, deg1, deg2, r2(bc21), r2(bc22),
                    Wm2a, r2(bm2a), Wm2b, r2(bm2b))

    m1, m2 = _sc_mean(idx1, idx2, h.reshape(2 * N_T, HALF), z128_hbm)
    wfb_pad = jnp.zeros((DIM, _OUTW), f32).at[:, :5].set(Wfb)
    bfb_pad = jnp.zeros((1, _OUTW), f32).at[0, :5].set(bfb)
    out = _tc_final(m1, m2, cnt1, cnt2, Wfa, r2(bfa), wfb_pad, bfb_pad)
    return out[:, :5]


# speedup vs baseline: 5.1710x; 1.1097x over previous
import functools

import jax
import jax.numpy as jnp
from jax import lax
from jax.experimental import pallas as pl
from jax.experimental.pallas import tpu as pltpu
from jax.experimental.pallas import tpu_sc as plsc

N_T = 10000
N_O = 1000
E = 160000
DIM = 256
FIN = 4652
HALF = 128

NC = 2
NS = 16
LANES = 16

CHUNK = 128
NSUB = 8
E_PAD = NS * CHUNK * 80
N_PAD = NS * CHUNK * NSUB

ACC_NT = 10240
ACC_NO = 1024
ZROWS = ACC_NT // NS


def _sc_mesh():
    return plsc.VectorSubcoreMesh(
        core_axis_name="c", subcore_axis_name="s", num_cores=NC,
        num_subcores=NS)


def _hist_body(col1, col2, idx1, idx2, ones_hbm, z128_hbm,
               deg1_out, deg2_out, cnt1_out, cnt2_out,
               acc_deg, acc_cnt, ones_v, zb, ib):
    c = lax.axis_index("c")
    t = lax.axis_index("s")

    pltpu.sync_copy(ones_hbm, ones_v)
    pltpu.sync_copy(z128_hbm, zb)
    for m in range(ZROWS // CHUNK):
        pltpu.sync_copy(zb, acc_deg.at[pl.ds(t * ZROWS + m * CHUNK, CHUNK)])
    pltpu.sync_copy(zb.at[pl.ds(0, ACC_NO // NS)],
                    acc_cnt.at[pl.ds(t * (ACC_NO // NS), ACC_NO // NS)])
    plsc.subcore_barrier()

    def _accum(src2d, acc, nchunks):
        def body(j, _):
            pltpu.sync_copy(src2d.at[pl.ds(t * nchunks + j * NSUB, NSUB)],
                            ib)
            for sub in range(NSUB):
                pltpu.sync_copy(ones_v, acc.at[ib.at[sub]], add=True)
            return 0
        lax.fori_loop(0, nchunks // NSUB, body, 0)

    @pl.when(c == 0)
    def _():
        _accum(col1, acc_deg, E_PAD // (NS * CHUNK))
        _accum(idx1, acc_cnt, N_PAD // (NS * CHUNK))

    @pl.when(c == 1)
    def _():
        _accum(col2, acc_deg, E_PAD // (NS * CHUNK))
        _accum(idx2, acc_cnt, N_PAD // (NS * CHUNK))

    plsc.subcore_barrier()

    def _dump(acc, out, base_rows, nch, active):
        @pl.when(active)
        def _():
            for m in range(nch):
                sl = pl.ds(base_rows + m * CHUNK, CHUNK)
                pltpu.sync_copy(acc.at[sl], zb)
                pltpu.sync_copy(zb, out.at[sl])

    @pl.when(c == 0)
    def _():
        _dump(acc_deg, deg1_out, t * ZROWS, ZROWS // CHUNK, t >= 0)
        _dump(acc_cnt, cnt1_out, t * CHUNK, 1, t < 8)

    @pl.when(c == 1)
    def _():
        _dump(acc_deg, deg2_out, t * ZROWS, ZROWS // CHUNK, t >= 0)
        _dump(acc_cnt, cnt2_out, t * CHUNK, 1, t < 8)


def _sc_hist(col1, col2, idx1, idx2, ones_hbm, z128_hbm):
    f = pl.kernel(
        _hist_body,
        out_type=[
            jax.ShapeDtypeStruct((ACC_NT, HALF), jnp.float32),
            jax.ShapeDtypeStruct((ACC_NT, HALF), jnp.float32),
            jax.ShapeDtypeStruct((ACC_NO, HALF), jnp.float32),
            jax.ShapeDtypeStruct((ACC_NO, HALF), jnp.float32),
        ],
        mesh=_sc_mesh(),
        scratch_types=[
            pltpu.VMEM_SHARED((ACC_NT, HALF), jnp.float32),
            pltpu.VMEM_SHARED((ACC_NO, HALF), jnp.float32),
            pltpu.VMEM((CHUNK, HALF), jnp.float32),
            pltpu.VMEM((CHUNK, HALF), jnp.float32),
            pltpu.VMEM((NSUB, CHUNK), jnp.int32),
        ],
    )
    return f(col1, col2, idx1, idx2, ones_hbm, z128_hbm)


DEPTH = 2


def _edge_body(row1, col1, row2, col2, a1v, a2v, z128_hbm,
               scat1_out, scat2_out,
               acc, rows, ibr, ibc, gb, *sems):
    c = lax.axis_index("c")
    t = lax.axis_index("s")
    nchunks = E_PAD // (NS * CHUNK)
    sg, ss = sems[:DEPTH], sems[DEPTH:]

    def run_list(row2d, col2d, a_hbm, out_hbm):
        pltpu.sync_copy(z128_hbm, rows.at[0])
        for m in range(ZROWS // CHUNK):
            pltpu.sync_copy(rows.at[0], acc.at[pl.ds(t * ZROWS + m * CHUNK,
                                                     CHUNK)])
        plsc.subcore_barrier()

        def body(j, _):
            sl = pl.ds(t * nchunks + j * NSUB, NSUB)
            pltpu.sync_copy(row2d.at[sl], ibr)
            pltpu.sync_copy(col2d.at[sl], ibc)
            for sub in range(NSUB):
                for v in range(CHUNK // LANES):
                    r = ibr[sub, pl.ds(v * LANES, LANES)]
                    gb[sub, pl.ds(v * LANES, LANES)] = r * 2 + c
            dg = [None] * NSUB
            dsc = [None] * NSUB
            for sub in range(DEPTH):
                dg[sub] = pltpu.async_copy(a_hbm.at[gb.at[sub]],
                                           rows.at[sub], sg[sub])
            for sub in range(NSUB):
                b = sub % DEPTH
                dg[sub].wait()
                dsc[sub] = pltpu.async_copy(rows.at[b], acc.at[ibc.at[sub]],
                                            ss[b], add=True)
                nxt = sub + DEPTH
                if nxt < NSUB:
                    dsc[sub].wait()
                    dg[nxt] = pltpu.async_copy(a_hbm.at[gb.at[nxt]],
                                               rows.at[b], sg[b])
            for sub in range(NSUB - DEPTH, NSUB):
                dsc[sub].wait()
            return 0

        lax.fori_loop(0, nchunks // NSUB, body, 0)
        plsc.subcore_barrier()

        for m in range(ZROWS // CHUNK):
            sl = pl.ds(t * ZROWS + m * CHUNK, CHUNK)
            pltpu.sync_copy(acc.at[sl], rows.at[0])
            pltpu.sync_copy(rows.at[0], out_hbm.at[c, sl])
        plsc.subcore_barrier()

    run_list(row1, col1, a1v, scat1_out)
    run_list(row2, col2, a2v, scat2_out)


def _sc_edge_scatter(row1, col1, row2, col2, a1v, a2v, z128_hbm):
    f = pl.kernel(
        _edge_body,
        out_type=[
            jax.ShapeDtypeStruct((NC, ACC_NT, HALF), jnp.float32),
            jax.ShapeDtypeStruct((NC, ACC_NT, HALF), jnp.float32),
        ],
        mesh=_sc_mesh(),
        scratch_types=[
            pltpu.VMEM_SHARED((ACC_NT, HALF), jnp.float32),
            pltpu.VMEM((DEPTH, CHUNK, HALF), jnp.float32),
            pltpu.VMEM((NSUB, CHUNK), jnp.int32),
            pltpu.VMEM((NSUB, CHUNK), jnp.int32),
            pltpu.VMEM((NSUB, CHUNK), jnp.int32),
        ] + [pltpu.SemaphoreType.DMA] * (2 * DEPTH),
    )
    return f(row1, col1, row2, col2, a1v, a2v, z128_hbm)


def _mean_body(idx1, idx2, hv, z128_hbm,
               sum1_out, sum2_out,
               acc1, acc2, rows, zb, ib1, ib2, gb, *sems):
    c = lax.axis_index("c")
    t = lax.axis_index("s")
    zrows = ACC_NO // NS
    sg, s1, s2 = sems[:DEPTH], sems[DEPTH:2 * DEPTH], sems[2 * DEPTH:]

    pltpu.sync_copy(z128_hbm.at[pl.ds(0, zrows)], zb)
    pltpu.sync_copy(zb, acc1.at[pl.ds(t * zrows, zrows)])
    pltpu.sync_copy(zb, acc2.at[pl.ds(t * zrows, zrows)])
    plsc.subcore_barrier()

    pltpu.sync_copy(idx1.at[pl.ds(t * NSUB, NSUB)], ib1)
    pltpu.sync_copy(idx2.at[pl.ds(t * NSUB, NSUB)], ib2)
    for sub in range(NSUB):
        base = (t * NSUB + sub) * CHUNK
        for v in range(CHUNK // LANES):
            n = base + v * LANES + lax.iota(jnp.int32, LANES)
            n = jnp.minimum(n, N_T - 1)
            gb[sub, pl.ds(v * LANES, LANES)] = n * 2 + c
    dg = [None] * NSUB
    d1 = [None] * NSUB
    d2 = [None] * NSUB
    for sub in range(DEPTH):
        dg[sub] = pltpu.async_copy(hv.at[gb.at[sub]], rows.at[sub], sg[sub])
    for sub in range(NSUB):
        b = sub % DEPTH
        dg[sub].wait()
        d1[sub] = pltpu.async_copy(rows.at[b], acc1.at[ib1.at[sub]], s1[b],
                                   add=True)
        d2[sub] = pltpu.async_copy(rows.at[b], acc2.at[ib2.at[sub]], s2[b],
                                   add=True)
        nxt = sub + DEPTH
        if nxt < NSUB:
            d1[sub].wait()
            d2[sub].wait()
            dg[nxt] = pltpu.async_copy(hv.at[gb.at[nxt]], rows.at[b], sg[b])
    for sub in range(NSUB - DEPTH, NSUB):
        d1[sub].wait()
        d2[sub].wait()
    plsc.subcore_barrier()

    @pl.when(t < 8)
    def _():
        sl = pl.ds(t * (ACC_NO // 8), ACC_NO // 8)
        pltpu.sync_copy(acc1.at[sl], rows.at[0])
        pltpu.sync_copy(rows.at[0], sum1_out.at[c, sl])

    @pl.when(t >= 8)
    def _():
        sl = pl.ds((t - 8) * (ACC_NO // 8), ACC_NO // 8)
        pltpu.sync_copy(acc2.at[sl], rows.at[0])
        pltpu.sync_copy(rows.at[0], sum2_out.at[c, sl])


def _sc_mean(idx1, idx2, hv, z128_hbm):
    f = pl.kernel(
        _mean_body,
        out_type=[
            jax.ShapeDtypeStruct((NC, ACC_NO, HALF), jnp.float32),
            jax.ShapeDtypeStruct((NC, ACC_NO, HALF), jnp.float32),
        ],
        mesh=_sc_mesh(),
        scratch_types=[
            pltpu.VMEM_SHARED((ACC_NO, HALF), jnp.float32),
            pltpu.VMEM_SHARED((ACC_NO, HALF), jnp.float32),
            pltpu.VMEM((DEPTH, CHUNK, HALF), jnp.float32),
            pltpu.VMEM((ACC_NO // NS, HALF), jnp.float32),
            pltpu.VMEM((NSUB, CHUNK), jnp.int32),
            pltpu.VMEM((NSUB, CHUNK), jnp.int32),
            pltpu.VMEM((NSUB, CHUNK), jnp.int32),
        ] + [pltpu.SemaphoreType.DMA] * (3 * DEPTH),
    )
    return f(idx1, idx2, hv, z128_hbm)


_RB = 400


def _mlp_in_tc(x_ref, w1_ref, b1_ref, w2_ref, b2_ref, o_ref):
    h = jnp.dot(x_ref[...], w1_ref[...], preferred_element_type=jnp.float32)
    h = jnp.maximum(h + b1_ref[...], 0.0)
    o_ref[...] = (jnp.dot(h, w2_ref[...], preferred_element_type=jnp.float32)
                  + b2_ref[...])


def _tc_mlp_in(x, w1, b1, w2, b2, interpret=False):
    grid = (N_T // _RB,)
    return pl.pallas_call(
        _mlp_in_tc,
        grid=grid,
        in_specs=[
            pl.BlockSpec((_RB, FIN), lambda i: (i, 0)),
            pl.BlockSpec((FIN, DIM), lambda i: (0, 0)),
            pl.BlockSpec((1, DIM), lambda i: (0, 0)),
            pl.BlockSpec((DIM, DIM), lambda i: (0, 0)),
            pl.BlockSpec((1, DIM), lambda i: (0, 0)),
        ],
        out_specs=pl.BlockSpec((_RB, DIM), lambda i: (i, 0)),
        out_shape=jax.ShapeDtypeStruct((N_T, DIM), jnp.float32),
        interpret=interpret,
    )(x, w1, b1, w2, b2)


def _branch_tc(h_ref, d1_ref, d2_ref, w1_ref, w2_ref, a1_ref, a2_ref):
    h = h_ref[...]
    dinv1 = lax.rsqrt(d1_ref[...][:, 0:1] + 1.0)
    dinv2 = lax.rsqrt(d2_ref[...][:, 0:1] + 1.0)
    a1_ref[...] = dinv1 * jnp.dot(h, w1_ref[...],
                                  preferred_element_type=jnp.float32)
    a2_ref[...] = dinv2 * jnp.dot(h, w2_ref[...],
                                  preferred_element_type=jnp.float32)


def _tc_branch(h, deg1, deg2, w1, w2, interpret=False):
    grid = (N_T // _RB,)
    return pl.pallas_call(
        _branch_tc,
        grid=grid,
        in_specs=[
            pl.BlockSpec((_RB, DIM), lambda i: (i, 0)),
            pl.BlockSpec((_RB, HALF), lambda i: (i, 0)),
            pl.BlockSpec((_RB, HALF), lambda i: (i, 0)),
            pl.BlockSpec((DIM, DIM), lambda i: (0, 0)),
            pl.BlockSpec((DIM, DIM), lambda i: (0, 0)),
        ],
        out_specs=[
            pl.BlockSpec((_RB, DIM), lambda i: (i, 0)),
            pl.BlockSpec((_RB, DIM), lambda i: (i, 0)),
        ],
        out_shape=[
            jax.ShapeDtypeStruct((N_T, DIM), jnp.float32),
            jax.ShapeDtypeStruct((N_T, DIM), jnp.float32),
        ],
        interpret=interpret,
    )(h, deg1, deg2, w1, w2)


def _combine_tc(s1l_ref, s1h_ref, s2l_ref, s2h_ref,
                a1l_ref, a1h_ref, a2l_ref, a2h_ref,
                d1_ref, d2_ref, bc1_ref, bc2_ref,
                wa_ref, ba_ref, wb_ref, bb_ref, o_ref):
    dinv1 = lax.rsqrt(d1_ref[...][:, 0:1] + 1.0)
    dinv2 = lax.rsqrt(d2_ref[...][:, 0:1] + 1.0)
    x1l = jnp.maximum(dinv1 * (s1l_ref[0] + a1l_ref[...])
                      + bc1_ref[...][:, :HALF], 0.0)
    x1h = jnp.maximum(dinv1 * (s1h_ref[0] + a1h_ref[...])
                      + bc1_ref[...][:, HALF:], 0.0)
    x2l = jnp.maximum(dinv2 * (s2l_ref[0] + a2l_ref[...])
                      + bc2_ref[...][:, :HALF], 0.0)
    x2h = jnp.maximum(dinv2 * (s2h_ref[0] + a2h_ref[...])
                      + bc2_ref[...][:, HALF:], 0.0)
    wa = wa_ref[...]
    pre = (jnp.dot(x1l, wa[0:HALF], preferred_element_type=jnp.float32)
           + jnp.dot(x1h, wa[HALF:DIM], preferred_element_type=jnp.float32)
           + jnp.dot(x2l, wa[DIM:DIM + HALF],
                     preferred_element_type=jnp.float32)
           + jnp.dot(x2h, wa[DIM + HALF:], preferred_element_type=jnp.float32)
           + ba_ref[...])
    pre = jnp.maximum(pre, 0.0)
    o_ref[...] = (jnp.dot(pre, wb_ref[...], preferred_element_type=jnp.float32)
                  + bb_ref[...])


def _tc_combine(s1, s2, a1, a2, deg1, deg2, bc1, bc2, wa, ba, wb, bb,
                interpret=False):
    grid = (N_T // _RB,)
    half_spec = [
        pl.BlockSpec((1, _RB, HALF), lambda i: (0, i, 0)),
        pl.BlockSpec((1, _RB, HALF), lambda i: (1, i, 0)),
    ]
    a_spec = [
        pl.BlockSpec((_RB, HALF), lambda i: (i, 0)),
        pl.BlockSpec((_RB, HALF), lambda i: (i, 1)),
    ]
    return pl.pallas_call(
        _combine_tc,
        grid=grid,
        in_specs=half_spec + half_spec + a_spec + a_spec + [
            pl.BlockSpec((_RB, HALF), lambda i: (i, 0)),
            pl.BlockSpec((_RB, HALF), lambda i: (i, 0)),
            pl.BlockSpec((1, DIM), lambda i: (0, 0)),
            pl.BlockSpec((1, DIM), lambda i: (0, 0)),
            pl.BlockSpec((2 * DIM, DIM), lambda i: (0, 0)),
            pl.BlockSpec((1, DIM), lambda i: (0, 0)),
            pl.BlockSpec((DIM, DIM), lambda i: (0, 0)),
            pl.BlockSpec((1, DIM), lambda i: (0, 0)),
        ],
        out_specs=pl.BlockSpec((_RB, DIM), lambda i: (i, 0)),
        out_shape=jax.ShapeDtypeStruct((N_T, DIM), jnp.float32),
        interpret=interpret,
    )(s1, s1, s2, s2, a1, a1, a2, a2, deg1, deg2, bc1, bc2, wa, ba, wb, bb)


_OUTW = 128


def _final_tc(m1l_ref, m1h_ref, m2l_ref, m2h_ref, c1_ref, c2_ref,
              wa_ref, ba_ref, wb_ref, bb_ref, o_ref):
    cnt1 = jnp.maximum(c1_ref[...][:, 0:1], 1.0)
    cnt2 = jnp.maximum(c2_ref[...][:, 0:1], 1.0)
    s1l = m1l_ref[0] / cnt1
    s1h = m1h_ref[0] / cnt1
    s2l = m2l_ref[0] / cnt2
    s2h = m2h_ref[0] / cnt2
    wa = wa_ref[...]
    pre = (jnp.dot(s1l, wa[0:HALF], preferred_element_type=jnp.float32)
           + jnp.dot(s1h, wa[HALF:DIM], preferred_element_type=jnp.float32)
           + jnp.dot(s2l, wa[DIM:DIM + HALF],
                     preferred_element_type=jnp.float32)
           + jnp.dot(s2h, wa[DIM + HALF:], preferred_element_type=jnp.float32)
           + ba_ref[...])
    pre = jnp.maximum(pre, 0.0)
    z = (jnp.dot(pre, wb_ref[...], preferred_element_type=jnp.float32)
         + bb_ref[...])
    lanes = lax.broadcasted_iota(jnp.int32, z.shape, 1)
    z = jnp.where(lanes < 5, z, -1e30)
    m = jnp.max(z, axis=1, keepdims=True)
    e = jnp.exp(z - m)
    s = jnp.sum(e, axis=1, keepdims=True)
    o_ref[...] = z - m - jnp.log(s)


def _tc_final(m1, m2, cnt1, cnt2, wa, ba, wb, bb, interpret=False):
    half_spec = [
        pl.BlockSpec((1, N_O, HALF), lambda i: (0, 0, 0)),
        pl.BlockSpec((1, N_O, HALF), lambda i: (1, 0, 0)),
    ]
    return pl.pallas_call(
        _final_tc,
        grid=(1,),
        in_specs=half_spec + half_spec + [
            pl.BlockSpec((N_O, HALF), lambda i: (0, 0)),
            pl.BlockSpec((N_O, HALF), lambda i: (0, 0)),
            pl.BlockSpec((2 * DIM, DIM), lambda i: (0, 0)),
            pl.BlockSpec((1, DIM), lambda i: (0, 0)),
            pl.BlockSpec((DIM, _OUTW), lambda i: (0, 0)),
            pl.BlockSpec((1, _OUTW), lambda i: (0, 0)),
        ],
        out_specs=pl.BlockSpec((N_O, _OUTW), lambda i: (0, 0)),
        out_shape=jax.ShapeDtypeStruct((N_O, _OUTW), jnp.float32),
        interpret=interpret,
    )(m1, m1, m2, m2, cnt1, cnt2, wa, ba, wb, bb)


def kernel(x, edge_index_1, edge_index_2, index_1, index_2, Wi1, bi1, Wi2,
           bi2, Wc11, bc11, Wc12, bc12, Wm1a, bm1a, Wm1b, bm1b, Wc21, bc21,
           Wc22, bc22, Wm2a, bm2a, Wm2b, bm2b, Wfa, bfa, Wfb, bfb):
    f32 = jnp.float32
    i32 = jnp.int32

    def pad_edges(ei):
        row = jnp.concatenate(
            [ei[0], jnp.zeros((E_PAD - E,), i32)]).reshape(E_PAD // CHUNK,
                                                           CHUNK)
        col = jnp.concatenate(
            [ei[1], jnp.full((E_PAD - E,), N_T, i32)]).reshape(
                E_PAD // CHUNK, CHUNK)
        return row, col

    row1, col1 = pad_edges(edge_index_1)
    row2, col2 = pad_edges(edge_index_2)
    idx1 = jnp.concatenate(
        [index_1, jnp.full((N_PAD - N_T,), N_O, i32)]).reshape(
            N_PAD // CHUNK, CHUNK)
    idx2 = jnp.concatenate(
        [index_2, jnp.full((N_PAD - N_T,), N_O, i32)]).reshape(
            N_PAD // CHUNK, CHUNK)

    ones_hbm = jnp.ones((CHUNK, HALF), f32)
    z128_hbm = jnp.zeros((CHUNK, HALF), f32)

    def r2(b):
        return b.reshape(1, DIM)

    deg1, deg2, cnt1, cnt2 = _sc_hist(col1, col2, idx1, idx2, ones_hbm,
                                      z128_hbm)

    h = _tc_mlp_in(x, Wi1, r2(bi1), Wi2, r2(bi2))

    a1, a2 = _tc_branch(h, deg1, deg2, Wc11, Wc12)
    s1, s2 = _sc_edge_scatter(row1, col1, row2, col2,
                              a1.reshape(2 * N_T, HALF),
                              a2.reshape(2 * N_T, HALF), z128_hbm)
    h = _tc_combine(s1, s2, a1, a2, deg1, deg2, r2(bc11), r2(bc12),
                    Wm1a, r2(bm1a), Wm1b, r2(bm1b))

    a1, a2 = _tc_branch(h, deg1, deg2, Wc21, Wc22)
    s1, s2 = _sc_edge_scatter(row1, col1, row2, col2,
                              a1.reshape(2 * N_T, HALF),
                              a2.reshape(2 * N_T, HALF), z128_hbm)
    h = _tc_combine(s1, s2, a1, a2, deg1, deg2, r2(bc21), r2(bc22),
                    Wm2a, r2(bm2a), Wm2b, r2(bm2b))

    m1, m2 = _sc_mean(idx1, idx2, h.reshape(2 * N_T, HALF), z128_hbm)
    wfb_pad = jnp.zeros((DIM, _OUTW), f32).at[:, :5].set(Wfb)
    bfb_pad = jnp.zeros((1, _OUTW), f32).at[0, :5].set(bfb)
    out = _tc_final(m1, m2, cnt1, cnt2, Wfa, r2(bfa), wfb_pad, bfb_pad)
    return out[:, :5]
